# Initial kernel scaffold; baseline (speedup 1.0000x reference)
import jax
import jax.numpy as jnp
from jax.experimental import pallas as pl


def kernel(x, edge_index, graph_ids, W1c, b1c, W2c, b2c, Wl1, bl1, Wl2, bl2):
    raise NotImplementedError("write your pallas kernel here")


# speedup vs baseline: 12.4603x; 12.4603x over previous
import functools

import jax
import jax.numpy as jnp
from jax import lax
from jax.experimental import pallas as pl
from jax.experimental.pallas import tpu as pltpu
from jax.experimental.pallas import tpu_sc as plsc

N = 10000
E = 160000
H = 256
B = 16

NUM_WORKERS = 32
CHUNK = 5008
E_PAD = NUM_WORKERS * CHUNK
NITER = CHUNK // 16

ROWS = 1000
GRID = N // ROWS


def _tc_main_body(x_ref, w1_ref, b1_ref, w2_ref, b2_ref, gid_ref,
                  wl1_ref, bl1_ref, wl2_ref, bl2_ref,
                  asg_ref, sel_ref, pos_ref, gemb_ref, s_ref, g_ref):
    i = pl.program_id(0)
    xb = x_ref[...]
    h = jnp.maximum(
        jnp.dot(xb, w1_ref[...], preferred_element_type=jnp.float32)
        + b1_ref[...], 0.0)
    logits = (jnp.dot(h, w2_ref[...], preferred_element_type=jnp.float32)
              + b2_ref[...])
    m = jnp.max(logits, axis=1, keepdims=True)
    e = jnp.exp(logits - m)
    a = e / jnp.sum(e, axis=1, keepdims=True)
    asg_ref[...] = a
    sel_ref[...] = (a[:, 0:1] >= a[:, 1:2]).astype(jnp.int32)

    onehot = jnp.where(
        gid_ref[...] == lax.broadcasted_iota(jnp.int32, (1, B), 1), 1.0, 0.0)
    ge = lax.dot_general(onehot, xb, (((0,), (0,)), ((), ())),
                         preferred_element_type=jnp.float32)
    pe = lax.dot_general(onehot, a[:, 0:1] * xb, (((0,), (0,)), ((), ())),
                         preferred_element_type=jnp.float32)

    @pl.when(i == 0)
    def _():
        pos_ref[...] = jnp.zeros_like(pos_ref)
        gemb_ref[...] = jnp.zeros_like(gemb_ref)

    pos_ref[...] += pe
    gemb_ref[...] += ge

    @pl.when(i == GRID - 1)
    def _():
        def head(emb):
            hh = jnp.maximum(
                jnp.dot(emb, wl1_ref[...], preferred_element_type=jnp.float32)
                + bl1_ref[...], 0.0)
            return (jnp.dot(hh, wl2_ref[...],
                            preferred_element_type=jnp.float32) + bl2_ref[...])
        s_ref[...] = head(pos_ref[...])
        g_ref[...] = head(gemb_ref[...])


_tc_main = pl.pallas_call(
    _tc_main_body,
    grid=(GRID,),
    in_specs=[
        pl.BlockSpec((ROWS, H), lambda i: (i, 0)),
        pl.BlockSpec((H, H), lambda i: (0, 0)),
        pl.BlockSpec((1, H), lambda i: (0, 0)),
        pl.BlockSpec((H, 2), lambda i: (0, 0)),
        pl.BlockSpec((1, 2), lambda i: (0, 0)),
        pl.BlockSpec((ROWS, 1), lambda i: (i, 0)),
        pl.BlockSpec((H, H), lambda i: (0, 0)),
        pl.BlockSpec((1, H), lambda i: (0, 0)),
        pl.BlockSpec((H, 2), lambda i: (0, 0)),
        pl.BlockSpec((1, 2), lambda i: (0, 0)),
    ],
    out_specs=[
        pl.BlockSpec((ROWS, 2), lambda i: (i, 0)),
        pl.BlockSpec((ROWS, 1), lambda i: (i, 0)),
        pl.BlockSpec((B, H), lambda i: (0, 0)),
        pl.BlockSpec((B, H), lambda i: (0, 0)),
        pl.BlockSpec((B, 2), lambda i: (0, 0)),
        pl.BlockSpec((B, 2), lambda i: (0, 0)),
    ],
    out_shape=[
        jax.ShapeDtypeStruct((N, 2), jnp.float32),
        jax.ShapeDtypeStruct((N, 1), jnp.int32),
        jax.ShapeDtypeStruct((B, H), jnp.float32),
        jax.ShapeDtypeStruct((B, H), jnp.float32),
        jax.ShapeDtypeStruct((B, 2), jnp.float32),
        jax.ShapeDtypeStruct((B, 2), jnp.float32),
    ],
)


_sc_mesh = plsc.VectorSubcoreMesh(core_axis_name="c", subcore_axis_name="s")


@functools.partial(
    pl.kernel,
    mesh=_sc_mesh,
    compiler_params=pltpu.CompilerParams(needs_layout_passes=False),
    out_type=jax.ShapeDtypeStruct((NUM_WORKERS, 64), jnp.float32),
    scratch_types=[
        pltpu.VMEM((2 * N,), jnp.float32),
        pltpu.VMEM((N,), jnp.int32),
        pltpu.VMEM((CHUNK,), jnp.int32),
        pltpu.VMEM((CHUNK,), jnp.int32),
        pltpu.VMEM((64,), jnp.float32),
    ],
)
def _sc_edges(asg_hbm, src_hbm, dst_hbm, gid_hbm, out_hbm,
              asg_v, gid_v, src_v, dst_v, acc_v):
    w = lax.axis_index("c") * 16 + lax.axis_index("s")
    pltpu.sync_copy(asg_hbm, asg_v)
    pltpu.sync_copy(gid_hbm, gid_v)
    pltpu.sync_copy(src_hbm.at[pl.ds(w * CHUNK, CHUNK)], src_v)
    pltpu.sync_copy(dst_hbm.at[pl.ds(w * CHUNK, CHUNK)], dst_v)
    for k in range(4):
        acc_v[pl.ds(k * 16, 16)] = jnp.zeros((16,), jnp.float32)

    base = w * CHUNK
    lane = lax.iota(jnp.int32, 16)

    def body(i, carry):
        off = pl.multiple_of(i * 16, 16)
        s16 = src_v[pl.ds(off, 16)]
        d16 = dst_v[pl.ds(off, 16)]
        as0 = plsc.load_gather(asg_v, [s16 * 2])
        as1 = plsc.load_gather(asg_v, [s16 * 2 + 1])
        ad0 = plsc.load_gather(asg_v, [d16 * 2])
        ad1 = plsc.load_gather(asg_v, [d16 * 2 + 1])
        g16 = plsc.load_gather(gid_v, [d16])
        mf = jnp.where(base + off + lane < E, 1.0, 0.0)
        slot = g16 * 4
        plsc.addupdate_scatter(acc_v, [slot], as0 * ad0 * mf)
        plsc.addupdate_scatter(acc_v, [slot + 1], as0 * ad1 * mf)
        plsc.addupdate_scatter(acc_v, [slot + 2], as1 * ad0 * mf)
        plsc.addupdate_scatter(acc_v, [slot + 3], as1 * ad1 * mf)
        return carry

    lax.fori_loop(0, NITER, body, 0)
    pltpu.sync_copy(acc_v, out_hbm.at[w])


def _tc_pen_body(p_ref, o_ref):
    S = jnp.sum(p_ref[...], axis=0)
    a00, a01 = S[:, 0:1], S[:, 1:2]
    a10, a11 = S[:, 2:3], S[:, 3:4]
    d0 = jnp.maximum(jnp.abs(a00) + jnp.abs(a01), 1e-5)
    d1 = jnp.maximum(jnp.abs(a10) + jnp.abs(a11), 1e-5)
    pen = (jnp.sum((a00 / d0 - 1.0) ** 2) + jnp.sum((a11 / d1 - 1.0) ** 2))
    o_ref[...] = jnp.reshape(pen / (2.0 * B), (1, 1))


_tc_pen = pl.pallas_call(
    _tc_pen_body,
    out_shape=jax.ShapeDtypeStruct((1, 1), jnp.float32),
)


def kernel(x, edge_index, graph_ids, W1c, b1c, W2c, b2c, Wl1, bl1, Wl2, bl2):
    asg, sel, posemb, gemb, s_out, g_out = _tc_main(
        x, W1c, b1c.reshape(1, H), W2c, b2c.reshape(1, 2),
        graph_ids.reshape(N, 1),
        Wl1, bl1.reshape(1, H), Wl2, bl2.reshape(1, 2))

    pad = jnp.zeros((E_PAD - E,), jnp.int32)
    src = jnp.concatenate([edge_index[0], pad])
    dst = jnp.concatenate([edge_index[1], pad])
    partials = _sc_edges(asg.reshape(2 * N), src, dst, graph_ids)

    pos_penalty = _tc_pen(partials.reshape(NUM_WORKERS, B, 4))[0, 0]

    return (s_out, g_out, posemb, gemb, pos_penalty, asg, x,
            sel.reshape(N))


# speedup vs baseline: 13.1933x; 1.0588x over previous
import functools

import jax
import jax.numpy as jnp
from jax import lax
from jax.experimental import pallas as pl
from jax.experimental.pallas import tpu as pltpu
from jax.experimental.pallas import tpu_sc as plsc

N = 10000
E = 160000
H = 256
B = 16

NUM_WORKERS = 32
CHUNK = E // NUM_WORKERS
NFULL = CHUNK // 16

ROWS = 2000
GRID = N // ROWS


def _tc_main_body(x_ref, w1_ref, b1_ref, w2_ref, b2_ref, gid_ref,
                  wl1_ref, bl1_ref, wl2_ref, bl2_ref,
                  asg_ref, a0_ref, sel_ref, pos_ref, gemb_ref, s_ref, g_ref):
    i = pl.program_id(0)
    xb = x_ref[...]
    h = jnp.maximum(
        jnp.dot(xb, w1_ref[...], preferred_element_type=jnp.float32)
        + b1_ref[...], 0.0)
    logits = (jnp.dot(h, w2_ref[...], preferred_element_type=jnp.float32)
              + b2_ref[...])
    m = jnp.max(logits, axis=1, keepdims=True)
    e = jnp.exp(logits - m)
    a = e / jnp.sum(e, axis=1, keepdims=True)
    asg_ref[...] = a
    a0_ref[...] = a[:, 0:1]
    sel_ref[...] = (a[:, 0:1] >= a[:, 1:2]).astype(jnp.int32)

    onehot = jnp.where(
        gid_ref[...] == lax.broadcasted_iota(jnp.int32, (1, B), 1), 1.0, 0.0)
    ge = lax.dot_general(onehot, xb, (((0,), (0,)), ((), ())),
                         preferred_element_type=jnp.float32)
    pe = lax.dot_general(onehot, a[:, 0:1] * xb, (((0,), (0,)), ((), ())),
                         preferred_element_type=jnp.float32)

    @pl.when(i == 0)
    def _():
        pos_ref[...] = jnp.zeros_like(pos_ref)
        gemb_ref[...] = jnp.zeros_like(gemb_ref)

    pos_ref[...] += pe
    gemb_ref[...] += ge

    @pl.when(i == GRID - 1)
    def _():
        def head(emb):
            hh = jnp.maximum(
                jnp.dot(emb, wl1_ref[...], preferred_element_type=jnp.float32)
                + bl1_ref[...], 0.0)
            return (jnp.dot(hh, wl2_ref[...],
                            preferred_element_type=jnp.float32) + bl2_ref[...])
        s_ref[...] = head(pos_ref[...])
        g_ref[...] = head(gemb_ref[...])


_tc_main = pl.pallas_call(
    _tc_main_body,
    grid=(GRID,),
    in_specs=[
        pl.BlockSpec((ROWS, H), lambda i: (i, 0)),
        pl.BlockSpec((H, H), lambda i: (0, 0)),
        pl.BlockSpec((1, H), lambda i: (0, 0)),
        pl.BlockSpec((H, 2), lambda i: (0, 0)),
        pl.BlockSpec((1, 2), lambda i: (0, 0)),
        pl.BlockSpec((ROWS, 1), lambda i: (i, 0)),
        pl.BlockSpec((H, H), lambda i: (0, 0)),
        pl.BlockSpec((1, H), lambda i: (0, 0)),
        pl.BlockSpec((H, 2), lambda i: (0, 0)),
        pl.BlockSpec((1, 2), lambda i: (0, 0)),
    ],
    out_specs=[
        pl.BlockSpec((ROWS, 2), lambda i: (i, 0)),
        pl.BlockSpec((ROWS, 1), lambda i: (i, 0)),
        pl.BlockSpec((ROWS, 1), lambda i: (i, 0)),
        pl.BlockSpec((B, H), lambda i: (0, 0)),
        pl.BlockSpec((B, H), lambda i: (0, 0)),
        pl.BlockSpec((B, 2), lambda i: (0, 0)),
        pl.BlockSpec((B, 2), lambda i: (0, 0)),
    ],
    out_shape=[
        jax.ShapeDtypeStruct((N, 2), jnp.float32),
        jax.ShapeDtypeStruct((N, 1), jnp.float32),
        jax.ShapeDtypeStruct((N, 1), jnp.int32),
        jax.ShapeDtypeStruct((B, H), jnp.float32),
        jax.ShapeDtypeStruct((B, H), jnp.float32),
        jax.ShapeDtypeStruct((B, 2), jnp.float32),
        jax.ShapeDtypeStruct((B, 2), jnp.float32),
    ],
)


_sc_mesh = plsc.VectorSubcoreMesh(core_axis_name="c", subcore_axis_name="s")


@functools.partial(
    pl.kernel,
    mesh=_sc_mesh,
    compiler_params=pltpu.CompilerParams(needs_layout_passes=False),
    out_type=jax.ShapeDtypeStruct((NUM_WORKERS, 64), jnp.float32),
    scratch_types=[
        pltpu.VMEM((N,), jnp.float32),
        pltpu.VMEM((N,), jnp.int32),
        pltpu.VMEM((CHUNK,), jnp.int32),
        pltpu.VMEM((CHUNK,), jnp.int32),
        pltpu.VMEM((16 * 64,), jnp.float32),
        pltpu.VMEM((64,), jnp.float32),
        pltpu.SemaphoreType.DMA,
    ],
)
def _sc_edges(a0_hbm, src_hbm, dst_hbm, gid_hbm, out_hbm,
              a0_v, gid_v, src_v, dst_v, acc_v, res_v, sem):
    w = lax.axis_index("c") * 16 + lax.axis_index("s")
    cp1 = pltpu.async_copy(a0_hbm, a0_v, sem)
    cp2 = pltpu.async_copy(gid_hbm, gid_v, sem)
    cp3 = pltpu.async_copy(src_hbm.at[pl.ds(w * CHUNK, CHUNK)], src_v, sem)
    cp4 = pltpu.async_copy(dst_hbm.at[pl.ds(w * CHUNK, CHUNK)], dst_v, sem)
    zero16f = jnp.zeros((16,), jnp.float32)
    for k in range(64):
        acc_v[pl.ds(k * 16, 16)] = zero16f
    cp1.wait()
    cp2.wait()
    cp3.wait()
    cp4.wait()

    lane = lax.iota(jnp.int32, 16)
    lane64 = lane * 64

    def step(off, mf):
        s16 = src_v[pl.ds(off, 16)]
        d16 = dst_v[pl.ds(off, 16)]
        as0 = plsc.load_gather(a0_v, [s16])
        ad0 = plsc.load_gather(a0_v, [d16])
        g16 = plsc.load_gather(gid_v, [d16])
        am = as0 * mf
        dm = ad0 * mf
        pm = as0 * dm
        base = lane64 + g16 * 4
        plsc.addupdate_scatter(acc_v, [base], pm)
        plsc.addupdate_scatter(acc_v, [base + 1], am - pm)
        plsc.addupdate_scatter(acc_v, [base + 2], dm - pm)
        plsc.addupdate_scatter(acc_v, [base + 3], mf - am - dm + pm)

    def body(i, carry):
        step(pl.multiple_of(i * 16, 16), jnp.full((16,), 1.0, jnp.float32))
        return carry

    lax.fori_loop(0, NFULL, body, 0)
    step(CHUNK - 16, jnp.where(lane >= 8, 1.0, 0.0))

    for j in range(4):
        t = acc_v[pl.ds(j * 16, 16)]
        for l in range(1, 16):
            t = t + acc_v[pl.ds(l * 64 + j * 16, 16)]
        res_v[pl.ds(j * 16, 16)] = t
    pltpu.sync_copy(res_v, out_hbm.at[w])


def _tc_pen_body(p_ref, o_ref):
    S = jnp.sum(p_ref[...], axis=0)
    a00, a01 = S[:, 0:1], S[:, 1:2]
    a10, a11 = S[:, 2:3], S[:, 3:4]
    d0 = jnp.maximum(jnp.abs(a00) + jnp.abs(a01), 1e-5)
    d1 = jnp.maximum(jnp.abs(a10) + jnp.abs(a11), 1e-5)
    pen = (jnp.sum((a00 / d0 - 1.0) ** 2) + jnp.sum((a11 / d1 - 1.0) ** 2))
    o_ref[...] = jnp.reshape(pen / (2.0 * B), (1, 1))


_tc_pen = pl.pallas_call(
    _tc_pen_body,
    out_shape=jax.ShapeDtypeStruct((1, 1), jnp.float32),
)


def kernel(x, edge_index, graph_ids, W1c, b1c, W2c, b2c, Wl1, bl1, Wl2, bl2):
    asg, a0, sel, posemb, gemb, s_out, g_out = _tc_main(
        x, W1c, b1c.reshape(1, H), W2c, b2c.reshape(1, 2),
        graph_ids.reshape(N, 1),
        Wl1, bl1.reshape(1, H), Wl2, bl2.reshape(1, 2))

    partials = _sc_edges(a0.reshape(N), edge_index[0], edge_index[1],
                         graph_ids)

    pos_penalty = _tc_pen(partials.reshape(NUM_WORKERS, B, 4))[0, 0]

    return (s_out, g_out, posemb, gemb, pos_penalty, asg, x,
            sel.reshape(N))


# speedup vs baseline: 16.4453x; 1.2465x over previous
import functools

import jax
import jax.numpy as jnp
from jax import lax
from jax.experimental import pallas as pl
from jax.experimental.pallas import tpu as pltpu
from jax.experimental.pallas import tpu_sc as plsc

N = 10000
E = 160000
H = 256
B = 16

NUM_WORKERS = 32
CHUNK = E // NUM_WORKERS
NFULL = CHUNK // 16
WIN = 5120

ROWS = 2048
GRID = (N + ROWS - 1) // ROWS
NPAD = GRID * ROWS


def _tc_main_body(x_ref, w1_ref, b1_ref, w2_ref, b2_ref, gid_ref,
                  wl1_ref, bl1_ref, wl2_ref, bl2_ref,
                  asgt_ref, a0_ref, sel_ref, pos_ref, gemb_ref, s_ref, g_ref):
    i = pl.program_id(0)
    valid = (lax.broadcasted_iota(jnp.int32, (ROWS, 1), 0) + i * ROWS) < N
    xb = jnp.where(valid, x_ref[...], 0.0)
    h = jnp.maximum(
        jnp.dot(xb, w1_ref[...], preferred_element_type=jnp.float32)
        + b1_ref[...], 0.0)
    logits = (jnp.dot(h, w2_ref[...], preferred_element_type=jnp.float32)
              + b2_ref[...])
    m = jnp.max(logits, axis=1, keepdims=True)
    e = jnp.exp(logits - m)
    a = e / jnp.sum(e, axis=1, keepdims=True)
    t0 = jnp.transpose(a[:, 0:1], (1, 0))
    t1 = jnp.transpose(a[:, 1:2], (1, 0))
    asgt_ref[...] = jnp.concatenate([t0, t1], axis=0)
    a0_ref[...] = t0
    sel_ref[...] = (t0 >= t1).astype(jnp.int32)

    onehot_t = jnp.where(
        gid_ref[...] == lax.broadcasted_iota(jnp.int32, (B, 1), 0), 1.0, 0.0)
    ge = lax.dot_general(onehot_t, xb, (((1,), (0,)), ((), ())),
                         preferred_element_type=jnp.float32)
    pe = lax.dot_general(onehot_t, a[:, 0:1] * xb, (((1,), (0,)), ((), ())),
                         preferred_element_type=jnp.float32)

    @pl.when(i == 0)
    def _():
        pos_ref[...] = jnp.zeros_like(pos_ref)
        gemb_ref[...] = jnp.zeros_like(gemb_ref)

    pos_ref[...] += pe
    gemb_ref[...] += ge

    @pl.when(i == GRID - 1)
    def _():
        def head(emb):
            hh = jnp.maximum(
                jnp.dot(emb, wl1_ref[...], preferred_element_type=jnp.float32)
                + bl1_ref[...], 0.0)
            return (jnp.dot(hh, wl2_ref[...],
                            preferred_element_type=jnp.float32) + bl2_ref[...])
        s_ref[...] = head(pos_ref[...])
        g_ref[...] = head(gemb_ref[...])


_tc_main = pl.pallas_call(
    _tc_main_body,
    grid=(GRID,),
    in_specs=[
        pl.BlockSpec((ROWS, H), lambda i: (i, 0)),
        pl.BlockSpec((H, H), lambda i: (0, 0)),
        pl.BlockSpec((1, H), lambda i: (0, 0)),
        pl.BlockSpec((H, 2), lambda i: (0, 0)),
        pl.BlockSpec((1, 2), lambda i: (0, 0)),
        pl.BlockSpec((1, ROWS), lambda i: (0, i)),
        pl.BlockSpec((H, H), lambda i: (0, 0)),
        pl.BlockSpec((1, H), lambda i: (0, 0)),
        pl.BlockSpec((H, 2), lambda i: (0, 0)),
        pl.BlockSpec((1, 2), lambda i: (0, 0)),
    ],
    out_specs=[
        pl.BlockSpec((2, ROWS), lambda i: (0, i)),
        pl.BlockSpec((1, ROWS), lambda i: (0, i)),
        pl.BlockSpec((1, ROWS), lambda i: (0, i)),
        pl.BlockSpec((B, H), lambda i: (0, 0)),
        pl.BlockSpec((B, H), lambda i: (0, 0)),
        pl.BlockSpec((B, 2), lambda i: (0, 0)),
        pl.BlockSpec((B, 2), lambda i: (0, 0)),
    ],
    out_shape=[
        jax.ShapeDtypeStruct((2, NPAD), jnp.float32),
        jax.ShapeDtypeStruct((1, NPAD), jnp.float32),
        jax.ShapeDtypeStruct((1, NPAD), jnp.int32),
        jax.ShapeDtypeStruct((B, H), jnp.float32),
        jax.ShapeDtypeStruct((B, H), jnp.float32),
        jax.ShapeDtypeStruct((B, 2), jnp.float32),
        jax.ShapeDtypeStruct((B, 2), jnp.float32),
    ],
)


_sc_mesh = plsc.VectorSubcoreMesh(core_axis_name="c", subcore_axis_name="s")


@functools.partial(
    pl.kernel,
    mesh=_sc_mesh,
    compiler_params=pltpu.CompilerParams(needs_layout_passes=False),
    out_type=jax.ShapeDtypeStruct((NUM_WORKERS, 64), jnp.float32),
    scratch_types=[
        pltpu.VMEM((NPAD,), jnp.float32),
        pltpu.VMEM((N,), jnp.int32),
        pltpu.VMEM((2, WIN), jnp.int32),
        pltpu.VMEM((16 * 64,), jnp.float32),
        pltpu.VMEM((64,), jnp.float32),
        pltpu.SemaphoreType.DMA,
    ],
)
def _sc_edges(a0_hbm, ei_hbm, gid_hbm, out_hbm,
              a0_v, gid_v, ei_v, acc_v, res_v, sem):
    w = lax.axis_index("c") * 16 + lax.axis_index("s")
    start = w * CHUNK
    base = (start // 128) * 128
    off_in = start - base
    cp1 = pltpu.async_copy(a0_hbm, a0_v, sem)
    cp2 = pltpu.async_copy(gid_hbm, gid_v, sem)
    cp3 = pltpu.async_copy(ei_hbm.at[:, pl.ds(base, WIN)], ei_v, sem)
    zero16f = jnp.zeros((16,), jnp.float32)
    for k in range(64):
        acc_v[pl.ds(k * 16, 16)] = zero16f
    cp1.wait()
    cp2.wait()
    cp3.wait()

    lane = lax.iota(jnp.int32, 16)
    lane64 = lane * 64
    z16 = jnp.zeros((16,), jnp.int32)
    o16 = jnp.full((16,), 1, jnp.int32)

    def step(off, mf):
        col = off_in + off + lane
        s16 = plsc.load_gather(ei_v, [z16, col])
        d16 = plsc.load_gather(ei_v, [o16, col])
        as0 = plsc.load_gather(a0_v, [s16])
        ad0 = plsc.load_gather(a0_v, [d16])
        g16 = plsc.load_gather(gid_v, [d16])
        am = as0 * mf
        dm = ad0 * mf
        pm = as0 * dm
        basev = lane64 + g16 * 4
        plsc.addupdate_scatter(acc_v, [basev], pm)
        plsc.addupdate_scatter(acc_v, [basev + 1], am - pm)
        plsc.addupdate_scatter(acc_v, [basev + 2], dm - pm)
        plsc.addupdate_scatter(acc_v, [basev + 3], mf - am - dm + pm)

    def body(i, carry):
        step(i * 16, jnp.full((16,), 1.0, jnp.float32))
        return carry

    lax.fori_loop(0, NFULL, body, 0)
    step(CHUNK - 16, jnp.where(lane >= 8, 1.0, 0.0))

    for j in range(4):
        t = acc_v[pl.ds(j * 16, 16)]
        for l in range(1, 16):
            t = t + acc_v[pl.ds(l * 64 + j * 16, 16)]
        res_v[pl.ds(j * 16, 16)] = t
    pltpu.sync_copy(res_v, out_hbm.at[w])


def _tc_pen_body(p_ref, o_ref):
    S = jnp.sum(p_ref[...], axis=0)
    a00, a01 = S[:, 0:1], S[:, 1:2]
    a10, a11 = S[:, 2:3], S[:, 3:4]
    d0 = jnp.maximum(jnp.abs(a00) + jnp.abs(a01), 1e-5)
    d1 = jnp.maximum(jnp.abs(a10) + jnp.abs(a11), 1e-5)
    pen = (jnp.sum((a00 / d0 - 1.0) ** 2) + jnp.sum((a11 / d1 - 1.0) ** 2))
    o_ref[...] = jnp.reshape(pen / (2.0 * B), (1, 1))


_tc_pen = pl.pallas_call(
    _tc_pen_body,
    out_shape=jax.ShapeDtypeStruct((1, 1), jnp.float32),
)


def kernel(x, edge_index, graph_ids, W1c, b1c, W2c, b2c, Wl1, bl1, Wl2, bl2):
    asgt, a0, sel, posemb, gemb, s_out, g_out = _tc_main(
        x, W1c, b1c.reshape(1, H), W2c, b2c.reshape(1, 2),
        graph_ids.reshape(1, N),
        Wl1, bl1.reshape(1, H), Wl2, bl2.reshape(1, 2))

    partials = _sc_edges(a0.reshape(NPAD), edge_index, graph_ids)

    pos_penalty = _tc_pen(partials.reshape(NUM_WORKERS, B, 4))[0, 0]

    return (s_out, g_out, posemb, gemb, pos_penalty,
            asgt[:, :N].T, x, sel[0, :N])


# speedup vs baseline: 21.6991x; 1.3195x over previous
import functools

import jax
import jax.numpy as jnp
from jax import lax
from jax.experimental import pallas as pl
from jax.experimental.pallas import tpu as pltpu
from jax.experimental.pallas import tpu_sc as plsc

N = 10000
E = 160000
H = 256
B = 16

NUM_WORKERS = 32
CHUNK = E // NUM_WORKERS
NFULL = CHUNK // 16
WIN = 5120

ROWS = 2048
GRID = (N + ROWS - 1) // ROWS
NPAD = GRID * ROWS


def _tc_main_body(x_ref, w1_ref, b1_ref, w2t_ref, b2t_ref, gid_ref,
                  wl1_ref, bl1_ref, wl2t_ref, bl2t_ref,
                  asgt_ref, a0_ref, sel_ref, pos_ref, gemb_ref, s_ref, g_ref):
    i = pl.program_id(0)
    valid = (lax.broadcasted_iota(jnp.int32, (ROWS, 1), 0) + i * ROWS) < N
    xb = jnp.where(valid, x_ref[...], 0.0)
    h = jnp.maximum(
        jnp.dot(xb, w1_ref[...], preferred_element_type=jnp.float32)
        + b1_ref[...], 0.0)
    lt = (lax.dot_general(w2t_ref[...], h, (((1,), (1,)), ((), ())),
                          preferred_element_type=jnp.float32) + b2t_ref[...])
    m = jnp.max(lt, axis=0, keepdims=True)
    e = jnp.exp(lt - m)
    at = e / jnp.sum(e, axis=0, keepdims=True)
    asgt_ref[...] = at
    a0row = at[0:1, :]
    a0_ref[...] = a0row
    sel_ref[...] = (a0row >= at[1:2, :]).astype(jnp.int32)

    onehot_t = jnp.where(
        gid_ref[...] == lax.broadcasted_iota(jnp.int32, (B, 1), 0), 1.0, 0.0)
    ge = lax.dot_general(onehot_t, xb, (((1,), (0,)), ((), ())),
                         preferred_element_type=jnp.float32)
    pe = lax.dot_general(onehot_t * a0row, xb, (((1,), (0,)), ((), ())),
                         preferred_element_type=jnp.float32)

    @pl.when(i == 0)
    def _():
        pos_ref[...] = jnp.zeros_like(pos_ref)
        gemb_ref[...] = jnp.zeros_like(gemb_ref)

    pos_ref[...] += pe
    gemb_ref[...] += ge

    @pl.when(i == GRID - 1)
    def _():
        def head_t(emb):
            hh = jnp.maximum(
                jnp.dot(emb, wl1_ref[...], preferred_element_type=jnp.float32)
                + bl1_ref[...], 0.0)
            return (lax.dot_general(wl2t_ref[...], hh,
                                    (((1,), (1,)), ((), ())),
                                    preferred_element_type=jnp.float32)
                    + bl2t_ref[...])
        s_ref[...] = head_t(pos_ref[...])
        g_ref[...] = head_t(gemb_ref[...])


_tc_main = pl.pallas_call(
    _tc_main_body,
    grid=(GRID,),
    in_specs=[
        pl.BlockSpec((ROWS, H), lambda i: (i, 0)),
        pl.BlockSpec((H, H), lambda i: (0, 0)),
        pl.BlockSpec((1, H), lambda i: (0, 0)),
        pl.BlockSpec((2, H), lambda i: (0, 0)),
        pl.BlockSpec((2, 1), lambda i: (0, 0)),
        pl.BlockSpec((1, ROWS), lambda i: (0, i)),
        pl.BlockSpec((H, H), lambda i: (0, 0)),
        pl.BlockSpec((1, H), lambda i: (0, 0)),
        pl.BlockSpec((2, H), lambda i: (0, 0)),
        pl.BlockSpec((2, 1), lambda i: (0, 0)),
    ],
    out_specs=[
        pl.BlockSpec((2, ROWS), lambda i: (0, i)),
        pl.BlockSpec((1, ROWS), lambda i: (0, i)),
        pl.BlockSpec((1, ROWS), lambda i: (0, i)),
        pl.BlockSpec((B, H), lambda i: (0, 0)),
        pl.BlockSpec((B, H), lambda i: (0, 0)),
        pl.BlockSpec((2, B), lambda i: (0, 0)),
        pl.BlockSpec((2, B), lambda i: (0, 0)),
    ],
    out_shape=[
        jax.ShapeDtypeStruct((2, NPAD), jnp.float32),
        jax.ShapeDtypeStruct((1, NPAD), jnp.float32),
        jax.ShapeDtypeStruct((1, NPAD), jnp.int32),
        jax.ShapeDtypeStruct((B, H), jnp.float32),
        jax.ShapeDtypeStruct((B, H), jnp.float32),
        jax.ShapeDtypeStruct((2, B), jnp.float32),
        jax.ShapeDtypeStruct((2, B), jnp.float32),
    ],
)


_sc_mesh = plsc.VectorSubcoreMesh(core_axis_name="c", subcore_axis_name="s")


@functools.partial(
    pl.kernel,
    mesh=_sc_mesh,
    compiler_params=pltpu.CompilerParams(needs_layout_passes=False),
    out_type=jax.ShapeDtypeStruct((NUM_WORKERS, 64), jnp.float32),
    scratch_types=[
        pltpu.VMEM((NPAD,), jnp.float32),
        pltpu.VMEM((N,), jnp.int32),
        pltpu.VMEM((2, WIN), jnp.int32),
        pltpu.VMEM((16 * 64,), jnp.float32),
        pltpu.VMEM((64,), jnp.float32),
        pltpu.SemaphoreType.DMA,
    ],
)
def _sc_edges(a0_hbm, ei_hbm, gid_hbm, out_hbm,
              a0_v, gid_v, ei_v, acc_v, res_v, sem):
    w = lax.axis_index("c") * 16 + lax.axis_index("s")
    start = w * CHUNK
    base = (start // 128) * 128
    off_in = start - base
    cp1 = pltpu.async_copy(a0_hbm, a0_v, sem)
    cp2 = pltpu.async_copy(gid_hbm, gid_v, sem)
    cp3 = pltpu.async_copy(ei_hbm.at[:, pl.ds(base, WIN)], ei_v, sem)
    zero16f = jnp.zeros((16,), jnp.float32)
    for k in range(64):
        acc_v[pl.ds(k * 16, 16)] = zero16f
    cp1.wait()
    cp2.wait()
    cp3.wait()

    lane = lax.iota(jnp.int32, 16)
    lane64 = lane * 64
    z16 = jnp.zeros((16,), jnp.int32)
    o16 = jnp.full((16,), 1, jnp.int32)

    def step(off, mf):
        col = off_in + off + lane
        s16 = plsc.load_gather(ei_v, [z16, col])
        d16 = plsc.load_gather(ei_v, [o16, col])
        as0 = plsc.load_gather(a0_v, [s16])
        ad0 = plsc.load_gather(a0_v, [d16])
        g16 = plsc.load_gather(gid_v, [d16])
        am = as0 * mf
        dm = ad0 * mf
        pm = as0 * dm
        basev = lane64 + g16 * 4
        plsc.addupdate_scatter(acc_v, [basev], pm)
        plsc.addupdate_scatter(acc_v, [basev + 1], am - pm)
        plsc.addupdate_scatter(acc_v, [basev + 2], dm - pm)
        plsc.addupdate_scatter(acc_v, [basev + 3], mf - am - dm + pm)

    def body(i, carry):
        step(i * 16, jnp.full((16,), 1.0, jnp.float32))
        return carry

    lax.fori_loop(0, NFULL, body, 0)
    step(CHUNK - 16, jnp.where(lane >= 8, 1.0, 0.0))

    for j in range(4):
        t = acc_v[pl.ds(j * 16, 16)]
        for l in range(1, 16):
            t = t + acc_v[pl.ds(l * 64 + j * 16, 16)]
        res_v[pl.ds(j * 16, 16)] = t
    pltpu.sync_copy(res_v, out_hbm.at[w])


def _tc_pen_body(p_ref, o_ref):
    S = jnp.sum(p_ref[...], axis=0)
    a00, a01 = S[:, 0:1], S[:, 1:2]
    a10, a11 = S[:, 2:3], S[:, 3:4]
    d0 = jnp.maximum(jnp.abs(a00) + jnp.abs(a01), 1e-5)
    d1 = jnp.maximum(jnp.abs(a10) + jnp.abs(a11), 1e-5)
    pen = (jnp.sum((a00 / d0 - 1.0) ** 2) + jnp.sum((a11 / d1 - 1.0) ** 2))
    o_ref[...] = jnp.reshape(pen / (2.0 * B), (1, 1))


_tc_pen = pl.pallas_call(
    _tc_pen_body,
    out_shape=jax.ShapeDtypeStruct((1, 1), jnp.float32),
)


def kernel(x, edge_index, graph_ids, W1c, b1c, W2c, b2c, Wl1, bl1, Wl2, bl2):
    asgt, a0, sel, posemb, gemb, s_out_t, g_out_t = _tc_main(
        x, W1c, b1c.reshape(1, H), W2c.T, b2c.reshape(2, 1),
        graph_ids.reshape(1, N),
        Wl1, bl1.reshape(1, H), Wl2.T, bl2.reshape(2, 1))

    partials = _sc_edges(a0.reshape(NPAD), edge_index, graph_ids)

    pos_penalty = _tc_pen(partials.reshape(NUM_WORKERS, B, 4))[0, 0]

    return (s_out_t.T, g_out_t.T, posemb, gemb, pos_penalty,
            asgt[:, :N].T, x, sel[0, :N])


# speedup vs baseline: 25.4244x; 1.1717x over previous
import functools

import jax
import jax.numpy as jnp
from jax import lax
from jax.experimental import pallas as pl
from jax.experimental.pallas import tpu as pltpu
from jax.experimental.pallas import tpu_sc as plsc

N = 10000
E = 160000
H = 256
B = 16

NUM_WORKERS = 32
CHUNK = E // NUM_WORKERS
NFULL = CHUNK // 16
WIN = 5120

ROWS = 2048
GRID = (N + ROWS - 1) // ROWS
NPAD = GRID * ROWS


def _tc_main_body(x_ref, w1_ref, b1_ref, w2t_ref, b2t_ref, gid_ref,
                  wl1_ref, bl1_ref, wl2t_ref, bl2t_ref,
                  asgt_ref, a0_ref, sel_ref, pos_ref, gemb_ref, s_ref, g_ref):
    i = pl.program_id(0)
    valid = (lax.broadcasted_iota(jnp.int32, (ROWS, 1), 0) + i * ROWS) < N
    xb = jnp.where(valid, x_ref[...], 0.0)
    h = jnp.maximum(
        jnp.dot(xb, w1_ref[...], preferred_element_type=jnp.float32)
        + b1_ref[...], 0.0)
    lt = (lax.dot_general(w2t_ref[...], h, (((1,), (1,)), ((), ())),
                          preferred_element_type=jnp.float32) + b2t_ref[...])
    m = jnp.max(lt, axis=0, keepdims=True)
    e = jnp.exp(lt - m)
    at = e / jnp.sum(e, axis=0, keepdims=True)
    asgt_ref[...] = at
    a0row = at[0:1, :]
    a0_ref[...] = a0row
    sel_ref[...] = (a0row >= at[1:2, :]).astype(jnp.int32)

    onehot_t = jnp.where(
        jnp.reshape(gid_ref[...], (1, ROWS))
        == lax.broadcasted_iota(jnp.int32, (B, 1), 0), 1.0, 0.0)
    ge = lax.dot_general(onehot_t, xb, (((1,), (0,)), ((), ())),
                         preferred_element_type=jnp.float32)
    pe = lax.dot_general(onehot_t * a0row, xb, (((1,), (0,)), ((), ())),
                         preferred_element_type=jnp.float32)

    @pl.when(i == 0)
    def _():
        pos_ref[...] = jnp.zeros_like(pos_ref)
        gemb_ref[...] = jnp.zeros_like(gemb_ref)

    pos_ref[...] += pe
    gemb_ref[...] += ge

    @pl.when(i == GRID - 1)
    def _():
        def head_t(emb):
            hh = jnp.maximum(
                jnp.dot(emb, wl1_ref[...], preferred_element_type=jnp.float32)
                + bl1_ref[...], 0.0)
            return (lax.dot_general(wl2t_ref[...], hh,
                                    (((1,), (1,)), ((), ())),
                                    preferred_element_type=jnp.float32)
                    + bl2t_ref[...])
        s_ref[...] = head_t(pos_ref[...])
        g_ref[...] = head_t(gemb_ref[...])


_tc_main = pl.pallas_call(
    _tc_main_body,
    grid=(GRID,),
    in_specs=[
        pl.BlockSpec((ROWS, H), lambda i: (i, 0)),
        pl.BlockSpec((H, H), lambda i: (0, 0)),
        pl.BlockSpec((1, H), lambda i: (0, 0)),
        pl.BlockSpec((2, H), lambda i: (0, 0)),
        pl.BlockSpec((2, 1), lambda i: (0, 0)),
        pl.BlockSpec((ROWS,), lambda i: (i,)),
        pl.BlockSpec((H, H), lambda i: (0, 0)),
        pl.BlockSpec((1, H), lambda i: (0, 0)),
        pl.BlockSpec((2, H), lambda i: (0, 0)),
        pl.BlockSpec((2, 1), lambda i: (0, 0)),
    ],
    out_specs=[
        pl.BlockSpec((2, ROWS), lambda i: (0, i)),
        pl.BlockSpec((1, ROWS), lambda i: (0, i)),
        pl.BlockSpec((1, ROWS), lambda i: (0, i)),
        pl.BlockSpec((B, H), lambda i: (0, 0)),
        pl.BlockSpec((B, H), lambda i: (0, 0)),
        pl.BlockSpec((2, B), lambda i: (0, 0)),
        pl.BlockSpec((2, B), lambda i: (0, 0)),
    ],
    out_shape=[
        jax.ShapeDtypeStruct((2, NPAD), jnp.float32),
        jax.ShapeDtypeStruct((1, NPAD), jnp.float32),
        jax.ShapeDtypeStruct((1, NPAD), jnp.int32),
        jax.ShapeDtypeStruct((B, H), jnp.float32),
        jax.ShapeDtypeStruct((B, H), jnp.float32),
        jax.ShapeDtypeStruct((2, B), jnp.float32),
        jax.ShapeDtypeStruct((2, B), jnp.float32),
    ],
)


_sc_mesh = plsc.VectorSubcoreMesh(core_axis_name="c", subcore_axis_name="s")


@functools.partial(
    pl.kernel,
    mesh=_sc_mesh,
    compiler_params=pltpu.CompilerParams(needs_layout_passes=False),
    out_type=jax.ShapeDtypeStruct((NUM_WORKERS, 64), jnp.float32),
    scratch_types=[
        pltpu.VMEM((NPAD,), jnp.float32),
        pltpu.VMEM((N,), jnp.int32),
        pltpu.VMEM((2, WIN), jnp.int32),
        pltpu.VMEM((16 * 64,), jnp.float32),
        pltpu.VMEM((64,), jnp.float32),
        pltpu.SemaphoreType.DMA,
    ],
)
def _sc_edges(a0_hbm, ei_hbm, gid_hbm, out_hbm,
              a0_v, gid_v, ei_v, acc_v, res_v, sem):
    w = lax.axis_index("c") * 16 + lax.axis_index("s")
    start = w * CHUNK
    base = (start // 128) * 128
    off_in = start - base
    cp1 = pltpu.async_copy(a0_hbm, a0_v, sem)
    cp2 = pltpu.async_copy(gid_hbm, gid_v, sem)
    cp3 = pltpu.async_copy(ei_hbm.at[:, pl.ds(base, WIN)], ei_v, sem)
    zero16f = jnp.zeros((16,), jnp.float32)
    for k in range(64):
        acc_v[pl.ds(k * 16, 16)] = zero16f
    cp1.wait()
    cp2.wait()
    cp3.wait()

    lane = lax.iota(jnp.int32, 16)
    lane64 = lane * 64
    z16 = jnp.zeros((16,), jnp.int32)
    o16 = jnp.full((16,), 1, jnp.int32)

    def step(off, mf):
        col = off_in + off + lane
        s16 = plsc.load_gather(ei_v, [z16, col])
        d16 = plsc.load_gather(ei_v, [o16, col])
        as0 = plsc.load_gather(a0_v, [s16])
        ad0 = plsc.load_gather(a0_v, [d16])
        g16 = plsc.load_gather(gid_v, [d16])
        am = as0 * mf
        dm = ad0 * mf
        pm = as0 * dm
        basev = lane64 + g16 * 4
        plsc.addupdate_scatter(acc_v, [basev], pm)
        plsc.addupdate_scatter(acc_v, [basev + 1], am - pm)
        plsc.addupdate_scatter(acc_v, [basev + 2], dm - pm)
        plsc.addupdate_scatter(acc_v, [basev + 3], mf - am - dm + pm)

    ones16 = jnp.full((16,), 1.0, jnp.float32)

    @plsc.parallel_loop(0, NFULL, unroll=4)
    def _loop(i):
        step(i * 16, ones16)
    step(CHUNK - 16, jnp.where(lane >= 8, 1.0, 0.0))

    for j in range(4):
        t = acc_v[pl.ds(j * 16, 16)]
        for l in range(1, 16):
            t = t + acc_v[pl.ds(l * 64 + j * 16, 16)]
        res_v[pl.ds(j * 16, 16)] = t
    pltpu.sync_copy(res_v, out_hbm.at[w])


def _tc_xcopy_body(x_ref, o_ref):
    o_ref[...] = x_ref[...]


_tc_xcopy = pl.pallas_call(
    _tc_xcopy_body,
    grid=(GRID,),
    in_specs=[pl.BlockSpec((ROWS, H), lambda i: (i, 0))],
    out_specs=pl.BlockSpec((ROWS, H), lambda i: (i, 0)),
    out_shape=jax.ShapeDtypeStruct((N, H), jnp.float32),
)


def _tc_pen_body(p_ref, o_ref):
    S = jnp.sum(p_ref[...], axis=0)
    a00, a01 = S[:, 0:1], S[:, 1:2]
    a10, a11 = S[:, 2:3], S[:, 3:4]
    d0 = jnp.maximum(jnp.abs(a00) + jnp.abs(a01), 1e-5)
    d1 = jnp.maximum(jnp.abs(a10) + jnp.abs(a11), 1e-5)
    pen = (jnp.sum((a00 / d0 - 1.0) ** 2) + jnp.sum((a11 / d1 - 1.0) ** 2))
    o_ref[...] = jnp.reshape(pen / (2.0 * B), (1, 1))


_tc_pen = pl.pallas_call(
    _tc_pen_body,
    out_shape=jax.ShapeDtypeStruct((1, 1), jnp.float32),
)


def kernel(x, edge_index, graph_ids, W1c, b1c, W2c, b2c, Wl1, bl1, Wl2, bl2):
    asgt, a0, sel, posemb, gemb, s_out_t, g_out_t = _tc_main(
        x, W1c, b1c.reshape(1, H), W2c.T, b2c.reshape(2, 1),
        graph_ids,
        Wl1, bl1.reshape(1, H), Wl2.T, bl2.reshape(2, 1))

    partials = _sc_edges(a0.reshape(NPAD), edge_index, graph_ids)

    pos_penalty = _tc_pen(partials.reshape(NUM_WORKERS, B, 4))[0, 0]

    return (s_out_t.T, g_out_t.T, posemb, gemb, pos_penalty,
            asgt[:, :N].T, _tc_xcopy(x), sel[0, :N])


# speedup vs baseline: 25.8672x; 1.0174x over previous
import functools

import jax
import jax.numpy as jnp
from jax import lax
from jax.experimental import pallas as pl
from jax.experimental.pallas import tpu as pltpu
from jax.experimental.pallas import tpu_sc as plsc

N = 10000
E = 160000
H = 256
B = 16

NUM_WORKERS = 32
CHUNK = E // NUM_WORKERS
NFULL = CHUNK // 16
WIN = 5120

ROWS = 2048
GRID = (N + ROWS - 1) // ROWS
NPAD = GRID * ROWS


def _tc_main_body(x_ref, w1_ref, b1_ref, w2t_ref, b2t_ref, gid_ref,
                  wl1_ref, bl1_ref, wl2t_ref, bl2t_ref,
                  asgt_ref, a0_ref, sel_ref, pos_ref, gemb_ref, s_ref, g_ref):
    i = pl.program_id(0)
    valid = (lax.broadcasted_iota(jnp.int32, (ROWS, 1), 0) + i * ROWS) < N
    xb = jnp.where(valid, x_ref[...], 0.0)
    h = jnp.maximum(
        jnp.dot(xb, w1_ref[...], preferred_element_type=jnp.float32)
        + b1_ref[...], 0.0)
    lt = (lax.dot_general(w2t_ref[...], h, (((1,), (1,)), ((), ())),
                          preferred_element_type=jnp.float32) + b2t_ref[...])
    m = jnp.max(lt, axis=0, keepdims=True)
    e = jnp.exp(lt - m)
    at = e / jnp.sum(e, axis=0, keepdims=True)
    asgt_ref[...] = at
    a0row = at[0:1, :]
    a0_ref[...] = a0row
    sel_ref[...] = (a0row >= at[1:2, :]).astype(jnp.int32)

    onehot_t = jnp.where(
        jnp.reshape(gid_ref[...], (1, ROWS))
        == lax.broadcasted_iota(jnp.int32, (B, 1), 0), 1.0, 0.0)
    ge = lax.dot_general(onehot_t, xb, (((1,), (0,)), ((), ())),
                         preferred_element_type=jnp.float32)
    pe = lax.dot_general(onehot_t * a0row, xb, (((1,), (0,)), ((), ())),
                         preferred_element_type=jnp.float32)

    @pl.when(i == 0)
    def _():
        pos_ref[...] = jnp.zeros_like(pos_ref)
        gemb_ref[...] = jnp.zeros_like(gemb_ref)

    pos_ref[...] += pe
    gemb_ref[...] += ge

    @pl.when(i == GRID - 1)
    def _():
        def head_t(emb):
            hh = jnp.maximum(
                jnp.dot(emb, wl1_ref[...], preferred_element_type=jnp.float32)
                + bl1_ref[...], 0.0)
            return (lax.dot_general(wl2t_ref[...], hh,
                                    (((1,), (1,)), ((), ())),
                                    preferred_element_type=jnp.float32)
                    + bl2t_ref[...])
        s_ref[...] = head_t(pos_ref[...])
        g_ref[...] = head_t(gemb_ref[...])


_tc_main = pl.pallas_call(
    _tc_main_body,
    grid=(GRID,),
    in_specs=[
        pl.BlockSpec((ROWS, H), lambda i: (i, 0)),
        pl.BlockSpec((H, H), lambda i: (0, 0)),
        pl.BlockSpec((1, H), lambda i: (0, 0)),
        pl.BlockSpec((2, H), lambda i: (0, 0)),
        pl.BlockSpec((2, 1), lambda i: (0, 0)),
        pl.BlockSpec((ROWS,), lambda i: (i,)),
        pl.BlockSpec((H, H), lambda i: (0, 0)),
        pl.BlockSpec((1, H), lambda i: (0, 0)),
        pl.BlockSpec((2, H), lambda i: (0, 0)),
        pl.BlockSpec((2, 1), lambda i: (0, 0)),
    ],
    out_specs=[
        pl.BlockSpec((2, ROWS), lambda i: (0, i)),
        pl.BlockSpec((1, ROWS), lambda i: (0, i)),
        pl.BlockSpec((1, ROWS), lambda i: (0, i)),
        pl.BlockSpec((B, H), lambda i: (0, 0)),
        pl.BlockSpec((B, H), lambda i: (0, 0)),
        pl.BlockSpec((2, B), lambda i: (0, 0)),
        pl.BlockSpec((2, B), lambda i: (0, 0)),
    ],
    out_shape=[
        jax.ShapeDtypeStruct((2, NPAD), jnp.float32),
        jax.ShapeDtypeStruct((1, NPAD), jnp.float32),
        jax.ShapeDtypeStruct((1, NPAD), jnp.int32),
        jax.ShapeDtypeStruct((B, H), jnp.float32),
        jax.ShapeDtypeStruct((B, H), jnp.float32),
        jax.ShapeDtypeStruct((2, B), jnp.float32),
        jax.ShapeDtypeStruct((2, B), jnp.float32),
    ],
)


_sc_mesh = plsc.VectorSubcoreMesh(core_axis_name="c", subcore_axis_name="s")


@functools.partial(
    pl.kernel,
    mesh=_sc_mesh,
    compiler_params=pltpu.CompilerParams(needs_layout_passes=False),
    out_type=jax.ShapeDtypeStruct((NUM_WORKERS, 64), jnp.float32),
    scratch_types=[
        pltpu.VMEM((NPAD,), jnp.float32),
        pltpu.VMEM((N,), jnp.int32),
        pltpu.VMEM((2, WIN), jnp.int32),
        pltpu.VMEM((16 * 65 + 16,), jnp.float32),
        pltpu.VMEM((64,), jnp.float32),
        pltpu.SemaphoreType.DMA,
    ],
)
def _sc_edges(a0_hbm, ei_hbm, gid_hbm, out_hbm,
              a0_v, gid_v, ei_v, acc_v, res_v, sem):
    w = lax.axis_index("c") * 16 + lax.axis_index("s")
    start = w * CHUNK
    base = (start // 128) * 128
    off_in = start - base
    cp1 = pltpu.async_copy(a0_hbm, a0_v, sem)
    cp2 = pltpu.async_copy(gid_hbm, gid_v, sem)
    cp3 = pltpu.async_copy(ei_hbm.at[:, pl.ds(base, WIN)], ei_v, sem)
    zero16f = jnp.zeros((16,), jnp.float32)
    for k in range(66):
        acc_v[pl.ds(k * 16, 16)] = zero16f
    cp1.wait()
    cp2.wait()
    cp3.wait()

    lane = lax.iota(jnp.int32, 16)
    lane65 = lane * 65
    z16 = jnp.zeros((16,), jnp.int32)
    o16 = jnp.full((16,), 1, jnp.int32)

    def step(off, mf):
        col = off_in + off + lane
        s16 = plsc.load_gather(ei_v, [z16, col])
        d16 = plsc.load_gather(ei_v, [o16, col])
        as0 = plsc.load_gather(a0_v, [s16])
        ad0 = plsc.load_gather(a0_v, [d16])
        g16 = plsc.load_gather(gid_v, [d16])
        am = as0 * mf
        dm = ad0 * mf
        pm = as0 * dm
        basev = lane65 + g16 * 4
        plsc.addupdate_scatter(acc_v, [basev], pm)
        plsc.addupdate_scatter(acc_v, [basev + 1], am - pm)
        plsc.addupdate_scatter(acc_v, [basev + 2], dm - pm)
        plsc.addupdate_scatter(acc_v, [basev + 3], mf - am - dm + pm)

    ones16 = jnp.full((16,), 1.0, jnp.float32)

    @plsc.parallel_loop(0, NFULL, unroll=4)
    def _loop(i):
        step(i * 16, ones16)
    step(CHUNK - 16, jnp.where(lane >= 8, 1.0, 0.0))

    for j in range(4):
        t = plsc.load_gather(acc_v, [j * 16 + lane])
        for l in range(1, 16):
            t = t + plsc.load_gather(acc_v, [l * 65 + j * 16 + lane])
        res_v[pl.ds(j * 16, 16)] = t
    pltpu.sync_copy(res_v, out_hbm.at[w])


def _tc_xcopy_body(x_ref, o_ref):
    o_ref[...] = x_ref[...]


_tc_xcopy = pl.pallas_call(
    _tc_xcopy_body,
    grid=(GRID,),
    in_specs=[pl.BlockSpec((ROWS, H), lambda i: (i, 0))],
    out_specs=pl.BlockSpec((ROWS, H), lambda i: (i, 0)),
    out_shape=jax.ShapeDtypeStruct((N, H), jnp.float32),
)


def _tc_pen_body(p_ref, o_ref):
    S = jnp.sum(p_ref[...], axis=0)
    a00, a01 = S[:, 0:1], S[:, 1:2]
    a10, a11 = S[:, 2:3], S[:, 3:4]
    d0 = jnp.maximum(jnp.abs(a00) + jnp.abs(a01), 1e-5)
    d1 = jnp.maximum(jnp.abs(a10) + jnp.abs(a11), 1e-5)
    pen = (jnp.sum((a00 / d0 - 1.0) ** 2) + jnp.sum((a11 / d1 - 1.0) ** 2))
    o_ref[...] = jnp.reshape(pen / (2.0 * B), (1, 1))


_tc_pen = pl.pallas_call(
    _tc_pen_body,
    out_shape=jax.ShapeDtypeStruct((1, 1), jnp.float32),
)


def kernel(x, edge_index, graph_ids, W1c, b1c, W2c, b2c, Wl1, bl1, Wl2, bl2):
    asgt, a0, sel, posemb, gemb, s_out_t, g_out_t = _tc_main(
        x, W1c, b1c.reshape(1, H), W2c.T, b2c.reshape(2, 1),
        graph_ids,
        Wl1, bl1.reshape(1, H), Wl2.T, bl2.reshape(2, 1))

    partials = _sc_edges(a0.reshape(NPAD), edge_index, graph_ids)

    pos_penalty = _tc_pen(partials.reshape(NUM_WORKERS, B, 4))[0, 0]

    return (s_out_t.T, g_out_t.T, posemb, gemb, pos_penalty,
            asgt[:, :N].T, _tc_xcopy(x), sel[0, :N])


# speedup vs baseline: 26.0702x; 1.0078x over previous
import functools

import jax
import jax.numpy as jnp
from jax import lax
from jax.experimental import pallas as pl
from jax.experimental.pallas import tpu as pltpu
from jax.experimental.pallas import tpu_sc as plsc

N = 10000
E = 160000
H = 256
B = 16

NUM_WORKERS = 32
CHUNK = E // NUM_WORKERS
NFULL = CHUNK // 16
WIN = 5120

ROWS = 2048
GRID = (N + ROWS - 1) // ROWS
NPAD = GRID * ROWS


def _tc_main_body(x_ref, w1_ref, b1_ref, w2t_ref, b2t_ref, gid_ref,
                  wl1_ref, bl1_ref, wl2t_ref, bl2t_ref,
                  asgt_ref, a0_ref, sel_ref, pos_ref, gemb_ref, s_ref, g_ref):
    i = pl.program_id(0)
    valid = (lax.broadcasted_iota(jnp.int32, (ROWS, 1), 0) + i * ROWS) < N
    xb = jnp.where(valid, x_ref[...], 0.0)
    h = jnp.maximum(
        jnp.dot(xb, w1_ref[...], preferred_element_type=jnp.float32)
        + b1_ref[...], 0.0)
    lt = (lax.dot_general(w2t_ref[...], h, (((1,), (1,)), ((), ())),
                          preferred_element_type=jnp.float32) + b2t_ref[...])
    m = jnp.max(lt, axis=0, keepdims=True)
    e = jnp.exp(lt - m)
    at = e / jnp.sum(e, axis=0, keepdims=True)
    asgt_ref[...] = at
    a0row = at[0:1, :]
    a0_ref[...] = a0row
    sel_ref[...] = (a0row >= at[1:2, :]).astype(jnp.int32)

    onehot_t = jnp.where(
        jnp.reshape(gid_ref[...], (1, ROWS))
        == lax.broadcasted_iota(jnp.int32, (B, 1), 0), 1.0, 0.0)
    ge = lax.dot_general(onehot_t, xb, (((1,), (0,)), ((), ())),
                         preferred_element_type=jnp.float32)
    pe = lax.dot_general(onehot_t * a0row, xb, (((1,), (0,)), ((), ())),
                         preferred_element_type=jnp.float32)

    @pl.when(i == 0)
    def _():
        pos_ref[...] = jnp.zeros_like(pos_ref)
        gemb_ref[...] = jnp.zeros_like(gemb_ref)

    pos_ref[...] += pe
    gemb_ref[...] += ge

    @pl.when(i == GRID - 1)
    def _():
        def head_t(emb):
            hh = jnp.maximum(
                jnp.dot(emb, wl1_ref[...], preferred_element_type=jnp.float32)
                + bl1_ref[...], 0.0)
            return (lax.dot_general(wl2t_ref[...], hh,
                                    (((1,), (1,)), ((), ())),
                                    preferred_element_type=jnp.float32)
                    + bl2t_ref[...])
        s_ref[...] = head_t(pos_ref[...])
        g_ref[...] = head_t(gemb_ref[...])


_tc_main = pl.pallas_call(
    _tc_main_body,
    grid=(GRID,),
    in_specs=[
        pl.BlockSpec((ROWS, H), lambda i: (i, 0)),
        pl.BlockSpec((H, H), lambda i: (0, 0)),
        pl.BlockSpec((1, H), lambda i: (0, 0)),
        pl.BlockSpec((2, H), lambda i: (0, 0)),
        pl.BlockSpec((2, 1), lambda i: (0, 0)),
        pl.BlockSpec((ROWS,), lambda i: (i,)),
        pl.BlockSpec((H, H), lambda i: (0, 0)),
        pl.BlockSpec((1, H), lambda i: (0, 0)),
        pl.BlockSpec((2, H), lambda i: (0, 0)),
        pl.BlockSpec((2, 1), lambda i: (0, 0)),
    ],
    out_specs=[
        pl.BlockSpec((2, ROWS), lambda i: (0, i)),
        pl.BlockSpec((1, ROWS), lambda i: (0, i)),
        pl.BlockSpec((1, ROWS), lambda i: (0, i)),
        pl.BlockSpec((B, H), lambda i: (0, 0)),
        pl.BlockSpec((B, H), lambda i: (0, 0)),
        pl.BlockSpec((2, B), lambda i: (0, 0)),
        pl.BlockSpec((2, B), lambda i: (0, 0)),
    ],
    out_shape=[
        jax.ShapeDtypeStruct((2, NPAD), jnp.float32),
        jax.ShapeDtypeStruct((1, NPAD), jnp.float32),
        jax.ShapeDtypeStruct((1, NPAD), jnp.int32),
        jax.ShapeDtypeStruct((B, H), jnp.float32),
        jax.ShapeDtypeStruct((B, H), jnp.float32),
        jax.ShapeDtypeStruct((2, B), jnp.float32),
        jax.ShapeDtypeStruct((2, B), jnp.float32),
    ],
)


_sc_mesh = plsc.VectorSubcoreMesh(core_axis_name="c", subcore_axis_name="s")


@functools.partial(
    pl.kernel,
    mesh=_sc_mesh,
    compiler_params=pltpu.CompilerParams(needs_layout_passes=False),
    out_type=jax.ShapeDtypeStruct((NUM_WORKERS, 64), jnp.float32),
    scratch_types=[
        pltpu.VMEM((NPAD,), jnp.float32),
        pltpu.VMEM((N,), jnp.int32),
        pltpu.VMEM((2, WIN), jnp.int32),
        pltpu.VMEM((16 * 65 + 16,), jnp.float32),
        pltpu.VMEM((64,), jnp.float32),
        pltpu.SemaphoreType.DMA,
    ],
)
def _sc_edges(a0_hbm, ei_hbm, gid_hbm, out_hbm,
              a0_v, gid_v, ei_v, acc_v, res_v, sem):
    w = lax.axis_index("c") * 16 + lax.axis_index("s")
    start = w * CHUNK
    base = (start // 128) * 128
    off_in = start - base
    cp1 = pltpu.async_copy(a0_hbm, a0_v, sem)
    cp2 = pltpu.async_copy(gid_hbm, gid_v, sem)
    cp3 = pltpu.async_copy(ei_hbm.at[:, pl.ds(base, WIN)], ei_v, sem)
    zero16f = jnp.zeros((16,), jnp.float32)

    def zbody(k, carry):
        acc_v[pl.ds(pl.multiple_of(k * 16, 16), 16)] = zero16f
        return carry

    lax.fori_loop(0, 66, zbody, 0)
    cp1.wait()
    cp2.wait()
    cp3.wait()

    lane = lax.iota(jnp.int32, 16)
    lane65 = lane * 65
    z16 = jnp.zeros((16,), jnp.int32)
    o16 = jnp.full((16,), 1, jnp.int32)

    def step(off, mf):
        col = off_in + off + lane
        s16 = plsc.load_gather(ei_v, [z16, col])
        d16 = plsc.load_gather(ei_v, [o16, col])
        as0 = plsc.load_gather(a0_v, [s16])
        ad0 = plsc.load_gather(a0_v, [d16])
        g16 = plsc.load_gather(gid_v, [d16])
        am = as0 * mf
        dm = ad0 * mf
        pm = as0 * dm
        basev = lane65 + g16 * 4
        plsc.addupdate_scatter(acc_v, [basev], pm)
        plsc.addupdate_scatter(acc_v, [basev + 1], am - pm)
        plsc.addupdate_scatter(acc_v, [basev + 2], dm - pm)
        plsc.addupdate_scatter(acc_v, [basev + 3], mf - am - dm + pm)

    ones16 = jnp.full((16,), 1.0, jnp.float32)

    @plsc.parallel_loop(0, NFULL, unroll=2)
    def _loop(i):
        step(i * 16, ones16)
    step(CHUNK - 16, jnp.where(lane >= 8, 1.0, 0.0))

    for j in range(4):
        def fbody(l, t):
            return t + plsc.load_gather(acc_v, [l * 65 + j * 16 + lane])
        t = lax.fori_loop(1, 16, fbody,
                          plsc.load_gather(acc_v, [j * 16 + lane]))
        res_v[pl.ds(j * 16, 16)] = t
    pltpu.sync_copy(res_v, out_hbm.at[w])


def _tc_xcopy_body(x_ref, o_ref):
    o_ref[...] = x_ref[...]


_tc_xcopy = pl.pallas_call(
    _tc_xcopy_body,
    grid=(GRID,),
    in_specs=[pl.BlockSpec((ROWS, H), lambda i: (i, 0))],
    out_specs=pl.BlockSpec((ROWS, H), lambda i: (i, 0)),
    out_shape=jax.ShapeDtypeStruct((N, H), jnp.float32),
)


def _tc_pen_body(p_ref, o_ref):
    S = jnp.sum(p_ref[...], axis=0)
    a00, a01 = S[:, 0:1], S[:, 1:2]
    a10, a11 = S[:, 2:3], S[:, 3:4]
    d0 = jnp.maximum(jnp.abs(a00) + jnp.abs(a01), 1e-5)
    d1 = jnp.maximum(jnp.abs(a10) + jnp.abs(a11), 1e-5)
    pen = (jnp.sum((a00 / d0 - 1.0) ** 2) + jnp.sum((a11 / d1 - 1.0) ** 2))
    o_ref[...] = jnp.reshape(pen / (2.0 * B), (1, 1))


_tc_pen = pl.pallas_call(
    _tc_pen_body,
    out_shape=jax.ShapeDtypeStruct((1, 1), jnp.float32),
)


def kernel(x, edge_index, graph_ids, W1c, b1c, W2c, b2c, Wl1, bl1, Wl2, bl2):
    asgt, a0, sel, posemb, gemb, s_out_t, g_out_t = _tc_main(
        x, W1c, b1c.reshape(1, H), W2c.T, b2c.reshape(2, 1),
        graph_ids,
        Wl1, bl1.reshape(1, H), Wl2.T, bl2.reshape(2, 1))

    partials = _sc_edges(a0.reshape(NPAD), edge_index, graph_ids)

    pos_penalty = _tc_pen(partials.reshape(NUM_WORKERS, B, 4))[0, 0]

    return (s_out_t.T, g_out_t.T, posemb, gemb, pos_penalty,
            asgt[:, :N].T, _tc_xcopy(x), sel[0, :N])
